# baseline (speedup 1.0000x reference)
import functools

import jax
import jax.numpy as jnp
from jax import lax
from jax.experimental import pallas as pl
from jax.experimental.pallas import tpu as pltpu
from jax.experimental.pallas import tpu_sc as plsc

_N = 10000
_NP = 10240
_E = 320000
_D = 128

_TILES = 16
_HE = _E // 2
_WA = 9984
_WL = _HE - 15 * _WA
_CHUNK = _NP // _TILES
_L = 16

_UNROLL_Z = 10
_UNROLL_S = 8


def _sc_mask_body(ei_hbm, mask_hbm, idx2_v, mask_v, red_v, blk_v, shared):
    c = lax.axis_index("c")
    s = lax.axis_index("s")
    col0 = c * _HE + s * _WA

    @pl.when(s < _TILES - 1)
    def _():
        pltpu.sync_copy(
            ei_hbm.at[:, pl.ds(col0, _WA)], idx2_v.at[:, pl.ds(0, _WA)]
        )

    @pl.when(s == _TILES - 1)
    def _():
        pltpu.sync_copy(ei_hbm.at[:, pl.ds(col0, _WL)], idx2_v)

    zeros = jnp.zeros((_L,), jnp.float32)
    ones = jnp.ones((_L,), jnp.float32)

    def zero_body(i, carry):
        for u in range(_UNROLL_Z):
            mask_v[pl.ds((i * _UNROLL_Z + u) * _L, _L)] = zeros
        return carry

    lax.fori_loop(0, _NP // (_L * _UNROLL_Z), zero_body, 0)

    def scat_body(i, carry):
        for u in range(_UNROLL_S):
            idx = idx2_v[1, pl.ds((i * _UNROLL_S + u) * _L, _L)]
            plsc.store_scatter(mask_v, [idx], ones)
        return carry

    n_outer = jnp.where(
        s == _TILES - 1, _WL // (_L * _UNROLL_S), _WA // (_L * _UNROLL_S)
    )
    lax.fori_loop(0, n_outer, scat_body, 0)

    pltpu.sync_copy(mask_v, shared.at[s])
    plsc.subcore_barrier()

    pltpu.sync_copy(shared.at[:, pl.ds(s * _CHUNK, _CHUNK)], blk_v)

    def red_body(j, carry):
        acc = blk_v[0, pl.ds(j * _L, _L)]
        for t in range(1, _TILES):
            acc = acc + blk_v[t, pl.ds(j * _L, _L)]
        red_v[pl.ds(j * _L, _L)] = jnp.where(acc > 0.0, ones, zeros)
        return carry

    lax.fori_loop(0, _CHUNK // _L, red_body, 0)

    pltpu.sync_copy(red_v, mask_hbm.at[pl.ds(c * _NP + s * _CHUNK, _CHUNK)])


def _sc_mask(edge_index):
    kern = functools.partial(
        pl.kernel,
        out_type=jax.ShapeDtypeStruct((2 * _NP,), jnp.float32),
        mesh=plsc.VectorSubcoreMesh(core_axis_name="c", subcore_axis_name="s"),
        compiler_params=pltpu.CompilerParams(needs_layout_passes=False),
        scratch_types=[
            pltpu.VMEM((2, _WL), jnp.int32),
            pltpu.VMEM((_NP,), jnp.float32),
            pltpu.VMEM((_CHUNK,), jnp.float32),
            pltpu.VMEM((_TILES, _CHUNK), jnp.float32),
            pltpu.VMEM_SHARED((_TILES, _NP), jnp.float32),
        ],
    )(_sc_mask_body)
    return kern(edge_index)


def _tc_body(x_ref, wv_ref, bv_ref, wo_ref, bo_ref, m_ref, o_ref):
    t = jnp.dot(x_ref[...], wv_ref[...], preferred_element_type=jnp.float32)
    t = (t + bv_ref[...]) * m_ref[...]
    o_ref[...] = (
        jnp.dot(t, wo_ref[...], preferred_element_type=jnp.float32) + bo_ref[...]
    )


_BLK = 10000


def _tc_epilogue(x, Wv, bv2, Wo, bo2, mask2):
    return pl.pallas_call(
        _tc_body,
        grid=(_N // _BLK,),
        in_specs=[
            pl.BlockSpec((_BLK, _D), lambda i: (i, 0)),
            pl.BlockSpec((_D, _D), lambda i: (0, 0)),
            pl.BlockSpec((1, _D), lambda i: (0, 0)),
            pl.BlockSpec((_D, _D), lambda i: (0, 0)),
            pl.BlockSpec((1, _D), lambda i: (0, 0)),
            pl.BlockSpec((_BLK, 1), lambda i: (i, 0)),
        ],
        out_specs=pl.BlockSpec((_BLK, _D), lambda i: (i, 0)),
        out_shape=jax.ShapeDtypeStruct((_N, _D), jnp.float32),
    )(x, Wv, bv2, Wo, bo2, mask2)


def kernel(x, edge_attr, edge_index, Wq, bq, Wk, bk, Wv, bv, Wb, bb, Wo, bo):
    mask_p = _sc_mask(edge_index)
    mask2 = jnp.maximum(mask_p[:_N], mask_p[_NP:_NP + _N]).reshape(_N, 1)
    bv2 = bv.reshape(1, _D)
    bo2 = bo.reshape(1, _D)
    return _tc_epilogue(x, Wv, bv2, Wo, bo2, mask2)


# speedup vs baseline: 1.0193x; 1.0193x over previous
import functools

import jax
import jax.numpy as jnp
from jax import lax
from jax.experimental import pallas as pl
from jax.experimental.pallas import tpu as pltpu
from jax.experimental.pallas import tpu_sc as plsc

_N = 10000
_NP = 10240
_E = 320000
_D = 128

_TILES = 16
_HE = _E // 2
_WA = 9984
_WL = _HE - 15 * _WA
_CHUNK = _NP // _TILES
_L = 16

_UNROLL_Z = 10
_UNROLL_S = 8


def _sc_mask_body(ei_hbm, mask_hbm, idx2_v, mask_v, red_v, blk_v, shared, sem):
    c = lax.axis_index("c")
    s = lax.axis_index("s")
    col0 = c * _HE + s * _WA
    cp = pltpu.async_copy(ei_hbm.at[:, pl.ds(col0, _WL)], idx2_v, sem)

    zeros = jnp.zeros((_L,), jnp.float32)
    ones = jnp.ones((_L,), jnp.float32)

    def zero_body(i, carry):
        for u in range(_UNROLL_Z):
            mask_v[pl.ds((i * _UNROLL_Z + u) * _L, _L)] = zeros
        return carry

    lax.fori_loop(0, _NP // (_L * _UNROLL_Z), zero_body, 0)
    cp.wait()

    def scat_body(i, carry):
        for u in range(_UNROLL_S):
            idx = idx2_v[1, pl.ds((i * _UNROLL_S + u) * _L, _L)]
            plsc.store_scatter(mask_v, [idx], ones)
        return carry

    lax.fori_loop(0, _WL // (_L * _UNROLL_S), scat_body, 0)

    pltpu.sync_copy(mask_v, shared.at[s])
    plsc.subcore_barrier()

    pltpu.sync_copy(shared.at[:, pl.ds(s * _CHUNK, _CHUNK)], blk_v)

    def red_body(j, carry):
        acc = blk_v[0, pl.ds(j * _L, _L)]
        for t in range(1, _TILES):
            acc = acc + blk_v[t, pl.ds(j * _L, _L)]
        red_v[pl.ds(j * _L, _L)] = jnp.where(acc > 0.0, ones, zeros)
        return carry

    lax.fori_loop(0, _CHUNK // _L, red_body, 0)

    pltpu.sync_copy(red_v, mask_hbm.at[pl.ds(c * _NP + s * _CHUNK, _CHUNK)])


def _sc_mask(edge_index):
    kern = functools.partial(
        pl.kernel,
        out_type=jax.ShapeDtypeStruct((2 * _NP,), jnp.float32),
        mesh=plsc.VectorSubcoreMesh(core_axis_name="c", subcore_axis_name="s"),
        compiler_params=pltpu.CompilerParams(needs_layout_passes=False),
        scratch_types=[
            pltpu.VMEM((2, _WL), jnp.int32),
            pltpu.VMEM((_NP,), jnp.float32),
            pltpu.VMEM((_CHUNK,), jnp.float32),
            pltpu.VMEM((_TILES, _CHUNK), jnp.float32),
            pltpu.VMEM_SHARED((_TILES, _NP), jnp.float32),
            pltpu.SemaphoreType.DMA,
        ],
    )(_sc_mask_body)
    return kern(edge_index)


def _tc_body(x_ref, wv_ref, bv_ref, wo_ref, bo_ref, m_ref, o_ref):
    t = jnp.dot(x_ref[...], wv_ref[...], preferred_element_type=jnp.float32)
    t = (t + bv_ref[...]) * m_ref[...]
    o_ref[...] = (
        jnp.dot(t, wo_ref[...], preferred_element_type=jnp.float32) + bo_ref[...]
    )


_BLK = 5000


def _tc_epilogue(x, Wv, bv2, Wo, bo2, mask2):
    return pl.pallas_call(
        _tc_body,
        grid=(_N // _BLK,),
        in_specs=[
            pl.BlockSpec((_BLK, _D), lambda i: (i, 0)),
            pl.BlockSpec((_D, _D), lambda i: (0, 0)),
            pl.BlockSpec((1, _D), lambda i: (0, 0)),
            pl.BlockSpec((_D, _D), lambda i: (0, 0)),
            pl.BlockSpec((1, _D), lambda i: (0, 0)),
            pl.BlockSpec((_BLK, 1), lambda i: (i, 0)),
        ],
        out_specs=pl.BlockSpec((_BLK, _D), lambda i: (i, 0)),
        out_shape=jax.ShapeDtypeStruct((_N, _D), jnp.float32),
    )(x, Wv, bv2, Wo, bo2, mask2)


def kernel(x, edge_attr, edge_index, Wq, bq, Wk, bk, Wv, bv, Wb, bb, Wo, bo):
    mask_p = _sc_mask(edge_index)
    mask2 = jnp.maximum(mask_p[:_N], mask_p[_NP:_NP + _N]).reshape(_N, 1)
    bv2 = bv.reshape(1, _D)
    bo2 = bo.reshape(1, _D)
    return _tc_epilogue(x, Wv, bv2, Wo, bo2, mask2)
